# baseline (speedup 1.0000x reference)
import jax
import jax.numpy as jnp
from jax.experimental import pallas as pl
from jax.experimental.pallas import tpu as pltpu

_EPS = 1e-5
_B, _C, _H, _W = 32, 8, 300, 300
_SCALES = (5, 10, 15)
_TS = 32
_NS = 9
_TAIL = _H - _NS * _TS
_SH = 16
_ROWS = 328
_BF = jnp.bfloat16


def _mm(a, b):
    return jax.lax.dot_general(a, b, (((1,), (0,)), ((), ())),
                               preferred_element_type=jnp.float32)


def _pool_mats(k):
    m = _H // k
    r = jax.lax.broadcasted_iota(jnp.int32, (_H, m), 0) // k
    c = jax.lax.broadcasted_iota(jnp.int32, (_H, m), 1)
    a = (r == c).astype(jnp.float32) * (1.0 / k)
    rt = jax.lax.broadcasted_iota(jnp.int32, (m, _H), 0)
    ct = jax.lax.broadcasted_iota(jnp.int32, (m, _H), 1) // k
    at = (rt == ct).astype(jnp.float32) * (1.0 / k)
    ut = (rt == ct).astype(jnp.float32)
    return a, at, ut


def _up_slab(p, k, ut, r0, ts):
    m = _H // k
    rows = jax.lax.broadcasted_iota(jnp.int32, (ts + 2, m), 0) + (r0 - 1)
    cols = jax.lax.broadcasted_iota(jnp.int32, (ts + 2, m), 1)
    udyn = (jnp.floor_divide(rows, k) == cols).astype(jnp.float32)
    uh = [_mm(udyn, p[c]) for c in range(_C)]
    uh2 = jnp.stack(uh, axis=0).reshape(_C * (ts + 2), m)
    up = _mm(uh2, ut).reshape(_C, ts + 2, _W)
    return jnp.pad(up, ((0, 0), (0, 0), (1, 1)))


def _conv9(wt, src, ts, nout):
    acc = None
    for dh in range(3):
        for dw in range(3):
            t = _mm(wt[3 * dh + dw], src[:, dh:dh + ts, dw:dw + _W])
            acc = t if acc is None else acc + t
    return acc


def _ria_kernel(x_ref, wxt_ref, wut_ref, wgt_ref, bx_ref, bg_ref,
                y_ref, s1_ref, s2_ref, xs_ref, cp_ref):
    x = x_ref[0]
    zc = lambda n, r, w: jnp.zeros((n, r, w), _BF)
    xs_ref[:, 0:_SH, :] = zc(_C, _SH, _W + 2)
    xs_ref[:, _H + _SH:_ROWS, :] = zc(_C, _ROWS - _H - _SH, _W + 2)
    xs_ref[:, _SH:_H + _SH, 0:1] = zc(_C, _H, 1)
    xs_ref[:, _SH:_H + _SH, _W + 1:_W + 2] = zc(_C, _H, 1)
    xs_ref[:, _SH:_H + _SH, 1:_W + 1] = x.astype(_BF)
    cp_ref[:, 0:_SH, :] = zc(24, _SH, _W + 2)
    cp_ref[:, _H + _SH:_ROWS, :] = zc(24, _ROWS - _H - _SH, _W + 2)
    cp_ref[:, _SH:_H + _SH, 0:1] = zc(24, _H, 1)
    cp_ref[:, _SH:_H + _SH, _W + 1:_W + 2] = zc(24, _H, 1)

    x2 = x.reshape(_C * _H, _W)
    pools, uts = [], []
    for k in _SCALES:
        a, at, ut = _pool_mats(k)
        t1 = _mm(x2, a).reshape(_C, _H, _H // k)
        pools.append(jnp.stack([_mm(at, t1[c]) for c in range(_C)], axis=0))
        uts.append(ut)

    wxt = wxt_ref[...]
    wut = wut_ref[...]
    wgt = wgt_ref[...]
    bx = bx_ref[...]
    bg = bg_ref[...]

    def conv1_strip(r0, xsub, ts):
        cx = _conv9(wxt, xsub, ts, 24) + bx[:, :, None]
        parts = []
        for i, k in enumerate(_SCALES):
            u16 = _up_slab(pools[i], k, uts[i], r0, ts).astype(_BF)
            parts.append(cx[8 * i:8 * i + 8] + _conv9(wut[i], u16, ts, 8))
        return jnp.concatenate(parts, axis=0).astype(_BF)

    def body1(s, carry):
        r0 = s * _TS
        slab = xs_ref[:, pl.ds(r0, 64), :]
        xsub = slab[:, _SH - 1:_SH - 1 + _TS + 2, :]
        cp_ref[:, pl.ds(r0 + _SH, _TS), 1:_W + 1] = conv1_strip(r0, xsub, _TS)
        return carry

    jax.lax.fori_loop(0, _NS, body1, 0)
    r0t = _NS * _TS
    xsub_t = xs_ref[:, r0t + _SH - 1:r0t + _SH + 1 + _TAIL, :]
    cp_ref[:, r0t + _SH:r0t + _SH + _TAIL, 1:_W + 1] = conv1_strip(
        r0t, xsub_t, _TAIL)

    def gated_strip(csub, ts):
        g = _conv9(wgt, csub, ts, 16) + bg[:, :, None]
        return g[:_C] * jax.nn.sigmoid(g[_C:])

    def body2(s, carry):
        s1, s2 = carry
        r0 = s * _TS
        slab = cp_ref[:, pl.ds(r0, 64), :]
        ystrip = gated_strip(slab[:, _SH - 1:_SH + 1 + _TS, :], _TS)
        y_ref[0, :, pl.ds(r0, _TS), :] = ystrip
        return (s1 + jnp.sum(ystrip, axis=1),
                s2 + jnp.sum(ystrip * ystrip, axis=1))

    z = jnp.zeros((_C, _W), jnp.float32)
    s1, s2 = jax.lax.fori_loop(0, _NS, body2, (z, z))
    csub_t = cp_ref[:, r0t + _SH - 1:r0t + _SH + 1 + _TAIL, :]
    ytail = gated_strip(csub_t, _TAIL)
    y_ref[0, :, r0t:r0t + _TAIL, :] = ytail
    s1_ref[0] = s1 + jnp.sum(ytail, axis=1)
    s2_ref[0] = s2 + jnp.sum(ytail * ytail, axis=1)


def _affine_kernel(y_ref, a_ref, b_ref, o_ref):
    o_ref[0] = y_ref[0] * a_ref[0] + b_ref[0]


def kernel(x, w5, b5, w10, b10, w15, b15, gw, gb, mw, mb, gamma, beta):
    f32 = jnp.float32
    wx_all = jnp.concatenate([w5[:, :8], w10[:, :8], w15[:, :8]], axis=0)
    wxt = jnp.transpose(wx_all, (2, 3, 0, 1)).reshape(9, 24, 8).astype(_BF)
    wu_all = jnp.stack([w5[:, 8:], w10[:, 8:], w15[:, 8:]], axis=0)
    wut = jnp.transpose(wu_all, (0, 3, 4, 1, 2)).reshape(3, 9, 8, 8)
    wut = wut.astype(_BF)
    wg_all = jnp.concatenate([gw, mw], axis=0)
    wgt = jnp.transpose(wg_all, (2, 3, 0, 1)).reshape(9, 16, 24).astype(_BF)
    bx = jnp.concatenate([b5, b10, b15], axis=0)[:, None]
    bg = jnp.concatenate([gb, mb], axis=0)[:, None]

    full = lambda shape: pl.BlockSpec(shape, lambda b: (0,) * len(shape))
    y, s1, s2 = pl.pallas_call(
        _ria_kernel,
        grid=(_B,),
        in_specs=[
            pl.BlockSpec((1, _C, _H, _W), lambda b: (b, 0, 0, 0)),
            full((9, 24, 8)), full((3, 9, 8, 8)), full((9, 16, 24)),
            full((24, 1)), full((16, 1)),
        ],
        out_specs=[
            pl.BlockSpec((1, _C, _H, _W), lambda b: (b, 0, 0, 0)),
            pl.BlockSpec((1, _C, _W), lambda b: (b, 0, 0)),
            pl.BlockSpec((1, _C, _W), lambda b: (b, 0, 0)),
        ],
        out_shape=[
            jax.ShapeDtypeStruct((_B, _C, _H, _W), f32),
            jax.ShapeDtypeStruct((_B, _C, _W), f32),
            jax.ShapeDtypeStruct((_B, _C, _W), f32),
        ],
        scratch_shapes=[
            pltpu.VMEM((_C, _ROWS, _W + 2), _BF),
            pltpu.VMEM((24, _ROWS, _W + 2), _BF),
        ],
        compiler_params=pltpu.CompilerParams(
            dimension_semantics=('parallel',)),
    )(x, wxt, wut, wgt, bx, bg)

    n = _B * _H * _W
    mean = jnp.sum(s1, axis=(0, 2)) / n
    var = jnp.sum(s2, axis=(0, 2)) / n - mean * mean
    scale = gamma * jax.lax.rsqrt(var + _EPS)
    shift = beta - mean * scale

    out = pl.pallas_call(
        _affine_kernel,
        grid=(_B,),
        in_specs=[
            pl.BlockSpec((1, _C, _H, _W), lambda b: (b, 0, 0, 0)),
            pl.BlockSpec((1, _C, 1, 1), lambda b: (0, 0, 0, 0)),
            pl.BlockSpec((1, _C, 1, 1), lambda b: (0, 0, 0, 0)),
        ],
        out_specs=pl.BlockSpec((1, _C, _H, _W), lambda b: (b, 0, 0, 0)),
        out_shape=jax.ShapeDtypeStruct((_B, _C, _H, _W), f32),
        compiler_params=pltpu.CompilerParams(
            dimension_semantics=('parallel',)),
    )(y, scale.reshape(1, _C, 1, 1), shift.reshape(1, _C, 1, 1))
    return out


# speedup vs baseline: 1.7756x; 1.7756x over previous
import jax
import jax.numpy as jnp
from jax.experimental import pallas as pl
from jax.experimental.pallas import tpu as pltpu

_EPS = 1e-5
_B, _C, _H, _W = 32, 8, 300, 300
_SCALES = (5, 10, 15)
_TS = 32
_NS = 9
_TAIL = _H - _NS * _TS
_SH = 16
_ROWS = 328
_BF = jnp.bfloat16


def _mm(a, b):
    return jax.lax.dot_general(a, b, (((1,), (0,)), ((), ())),
                               preferred_element_type=jnp.float32)


def _pool_mats(k):
    m = _H // k
    r = jax.lax.broadcasted_iota(jnp.int32, (_H, m), 0) // k
    c = jax.lax.broadcasted_iota(jnp.int32, (_H, m), 1)
    a = (r == c).astype(jnp.float32) * (1.0 / k)
    rt = jax.lax.broadcasted_iota(jnp.int32, (m, _H), 0)
    ct = jax.lax.broadcasted_iota(jnp.int32, (m, _H), 1) // k
    at = (rt == ct).astype(jnp.float32) * (1.0 / k)
    ut = (rt == ct).astype(jnp.float32)
    return a, at, ut


def _up_slab(p, k, ut, r0, ts):
    m = _H // k
    rows = jax.lax.broadcasted_iota(jnp.int32, (ts + 2, m), 0) + (r0 - 1)
    cols = jax.lax.broadcasted_iota(jnp.int32, (ts + 2, m), 1)
    udyn = (jnp.floor_divide(rows, k) == cols).astype(jnp.float32)
    uh = [_mm(udyn, p[c]) for c in range(_C)]
    uh2 = jnp.stack(uh, axis=0).reshape(_C * (ts + 2), m)
    up = _mm(uh2, ut).reshape(_C, ts + 2, _W)
    return jnp.pad(up, ((0, 0), (0, 0), (1, 1)))


def _conv9(w2d, src, ts, nout):
    p = jnp.concatenate(
        [src[:, dh:dh + ts, dw:dw + _W]
         for dh in range(3) for dw in range(3)], axis=0)
    return _mm(w2d, p)


def _ria_kernel(x_ref, wxt_ref, wut_ref, wgt_ref, bx_ref, bg_ref,
                y_ref, s1_ref, s2_ref, xs_ref, cp_ref):
    x = x_ref[0]
    zc = lambda n, r, w: jnp.zeros((n, r, w), _BF)
    xs_ref[:, 0:_SH, :] = zc(_C, _SH, _W + 2)
    xs_ref[:, _H + _SH:_ROWS, :] = zc(_C, _ROWS - _H - _SH, _W + 2)
    xs_ref[:, _SH:_H + _SH, 0:1] = zc(_C, _H, 1)
    xs_ref[:, _SH:_H + _SH, _W + 1:_W + 2] = zc(_C, _H, 1)
    xs_ref[:, _SH:_H + _SH, 1:_W + 1] = x.astype(_BF)
    cp_ref[:, 0:_SH, :] = zc(24, _SH, _W + 2)
    cp_ref[:, _H + _SH:_ROWS, :] = zc(24, _ROWS - _H - _SH, _W + 2)
    cp_ref[:, _SH:_H + _SH, 0:1] = zc(24, _H, 1)
    cp_ref[:, _SH:_H + _SH, _W + 1:_W + 2] = zc(24, _H, 1)

    x2 = x.reshape(_C * _H, _W)
    pools, uts = [], []
    for k in _SCALES:
        a, at, ut = _pool_mats(k)
        t1 = _mm(x2, a).reshape(_C, _H, _H // k)
        pools.append(jnp.stack([_mm(at, t1[c]) for c in range(_C)], axis=0))
        uts.append(ut)

    wxt = wxt_ref[...]
    wut = wut_ref[...]
    wgt = wgt_ref[...]
    bx = bx_ref[...]
    bg = bg_ref[...]

    def conv1_strip(r0, xsub, ts):
        cx = _conv9(wxt, xsub, ts, 24) + bx[:, :, None]
        parts = []
        for i, k in enumerate(_SCALES):
            u16 = _up_slab(pools[i], k, uts[i], r0, ts).astype(_BF)
            parts.append(cx[8 * i:8 * i + 8] + _conv9(wut[i], u16, ts, 8))
        return jnp.concatenate(parts, axis=0).astype(_BF)

    def body1(s, carry):
        r0 = s * _TS
        slab = xs_ref[:, pl.ds(r0, 64), :]
        xsub = slab[:, _SH - 1:_SH - 1 + _TS + 2, :]
        cp_ref[:, pl.ds(r0 + _SH, _TS), 1:_W + 1] = conv1_strip(r0, xsub, _TS)
        return carry

    jax.lax.fori_loop(0, _NS, body1, 0)
    r0t = _NS * _TS
    xsub_t = xs_ref[:, r0t + _SH - 1:r0t + _SH + 1 + _TAIL, :]
    cp_ref[:, r0t + _SH:r0t + _SH + _TAIL, 1:_W + 1] = conv1_strip(
        r0t, xsub_t, _TAIL)

    def gated_strip(csub, ts):
        g = _conv9(wgt, csub, ts, 16) + bg[:, :, None]
        return g[:_C] * jax.nn.sigmoid(g[_C:])

    def body2(s, carry):
        s1, s2 = carry
        r0 = s * _TS
        slab = cp_ref[:, pl.ds(r0, 64), :]
        ystrip = gated_strip(slab[:, _SH - 1:_SH + 1 + _TS, :], _TS)
        y_ref[0, :, pl.ds(r0, _TS), :] = ystrip
        return (s1 + jnp.sum(ystrip, axis=1),
                s2 + jnp.sum(ystrip * ystrip, axis=1))

    z = jnp.zeros((_C, _W), jnp.float32)
    s1, s2 = jax.lax.fori_loop(0, _NS, body2, (z, z))
    csub_t = cp_ref[:, r0t + _SH - 1:r0t + _SH + 1 + _TAIL, :]
    ytail = gated_strip(csub_t, _TAIL)
    y_ref[0, :, r0t:r0t + _TAIL, :] = ytail
    s1_ref[0] = s1 + jnp.sum(ytail, axis=1)
    s2_ref[0] = s2 + jnp.sum(ytail * ytail, axis=1)


def _affine_kernel(y_ref, a_ref, b_ref, o_ref):
    o_ref[0] = y_ref[0] * a_ref[0] + b_ref[0]


def kernel(x, w5, b5, w10, b10, w15, b15, gw, gb, mw, mb, gamma, beta):
    f32 = jnp.float32
    wx_all = jnp.concatenate([w5[:, :8], w10[:, :8], w15[:, :8]], axis=0)
    wxt = jnp.transpose(wx_all, (0, 2, 3, 1)).reshape(24, 72).astype(_BF)
    wu_all = jnp.stack([w5[:, 8:], w10[:, 8:], w15[:, 8:]], axis=0)
    wut = jnp.transpose(wu_all, (0, 1, 3, 4, 2)).reshape(3, 8, 72)
    wut = wut.astype(_BF)
    wg_all = jnp.concatenate([gw, mw], axis=0)
    wgt = jnp.transpose(wg_all, (0, 2, 3, 1)).reshape(16, 216).astype(_BF)
    bx = jnp.concatenate([b5, b10, b15], axis=0)[:, None]
    bg = jnp.concatenate([gb, mb], axis=0)[:, None]

    full = lambda shape: pl.BlockSpec(shape, lambda b: (0,) * len(shape))
    y, s1, s2 = pl.pallas_call(
        _ria_kernel,
        grid=(_B,),
        in_specs=[
            pl.BlockSpec((1, _C, _H, _W), lambda b: (b, 0, 0, 0)),
            full((24, 72)), full((3, 8, 72)), full((16, 216)),
            full((24, 1)), full((16, 1)),
        ],
        out_specs=[
            pl.BlockSpec((1, _C, _H, _W), lambda b: (b, 0, 0, 0)),
            pl.BlockSpec((1, _C, _W), lambda b: (b, 0, 0)),
            pl.BlockSpec((1, _C, _W), lambda b: (b, 0, 0)),
        ],
        out_shape=[
            jax.ShapeDtypeStruct((_B, _C, _H, _W), f32),
            jax.ShapeDtypeStruct((_B, _C, _W), f32),
            jax.ShapeDtypeStruct((_B, _C, _W), f32),
        ],
        scratch_shapes=[
            pltpu.VMEM((_C, _ROWS, _W + 2), _BF),
            pltpu.VMEM((24, _ROWS, _W + 2), _BF),
        ],
        compiler_params=pltpu.CompilerParams(
            dimension_semantics=('parallel',)),
    )(x, wxt, wut, wgt, bx, bg)

    n = _B * _H * _W
    mean = jnp.sum(s1, axis=(0, 2)) / n
    var = jnp.sum(s2, axis=(0, 2)) / n - mean * mean
    scale = gamma * jax.lax.rsqrt(var + _EPS)
    shift = beta - mean * scale

    out = pl.pallas_call(
        _affine_kernel,
        grid=(_B,),
        in_specs=[
            pl.BlockSpec((1, _C, _H, _W), lambda b: (b, 0, 0, 0)),
            pl.BlockSpec((1, _C, 1, 1), lambda b: (0, 0, 0, 0)),
            pl.BlockSpec((1, _C, 1, 1), lambda b: (0, 0, 0, 0)),
        ],
        out_specs=pl.BlockSpec((1, _C, _H, _W), lambda b: (b, 0, 0, 0)),
        out_shape=jax.ShapeDtypeStruct((_B, _C, _H, _W), f32),
        compiler_params=pltpu.CompilerParams(
            dimension_semantics=('parallel',)),
    )(y, scale.reshape(1, _C, 1, 1), shift.reshape(1, _C, 1, 1))
    return out
